# baseline (speedup 1.0000x reference)
import functools

import jax
import jax.numpy as jnp
from jax import lax
from jax.experimental import pallas as pl
from jax.experimental.pallas import tpu as pltpu
from jax.experimental.pallas import tpu_sc as plsc

N_NODES = 100000
N_NEIGH = 64
L = 16
NC, NS = 2, 16
NW = NC * NS
CH = 400
NCHUNKS = N_NODES // CH
MAXC = pl.cdiv(NCHUNKS, NW)


def _sc_body(x_hbm, w_hbm, b_hbm, o_hbm, xv, wv, bv, ov):
    wid = lax.axis_index("s") * NC + lax.axis_index("c")
    lane = lax.iota(jnp.int32, L)

    def chunk_body(i, carry):
        k = wid + i * NW

        @pl.when(k < NCHUNKS)
        def _():
            base = k * CH
            pltpu.sync_copy(x_hbm.at[pl.ds(base, CH)], xv)
            pltpu.sync_copy(w_hbm.at[pl.ds(base, CH)], wv)
            pltpu.sync_copy(b_hbm.at[pl.ds(base, CH)], bv)

            def row_iter(it, c2):
                r0 = it * L
                rows = r0 + lane
                accs = [None] * 4
                for j in range(N_NEIGH):
                    cols = jnp.full((L,), j, jnp.int32)
                    prod = plsc.load_gather(xv, [rows, cols]) * plsc.load_gather(
                        wv, [rows, cols]
                    )
                    a = j % 4
                    accs[a] = prod if accs[a] is None else accs[a] + prod
                vec = bv[pl.ds(r0, L)] + ((accs[0] + accs[1]) + (accs[2] + accs[3]))
                ov[pl.ds(r0, L)] = vec
                return c2

            lax.fori_loop(0, CH // L, row_iter, 0)
            pltpu.sync_copy(ov, o_hbm.at[pl.ds(base, CH)])

        return carry

    lax.fori_loop(0, MAXC, chunk_body, 0)


@jax.jit
def kernel(layer_input, weight, bias):
    mesh = plsc.VectorSubcoreMesh(core_axis_name="c", subcore_axis_name="s")
    run = pl.kernel(
        _sc_body,
        out_type=jax.ShapeDtypeStruct((N_NODES,), jnp.float32),
        mesh=mesh,
        compiler_params=pltpu.CompilerParams(
            needs_layout_passes=False,
            use_tc_tiling_on_sc=False,
        ),
        scratch_types=[
            pltpu.VMEM((CH, N_NEIGH), jnp.float32),
            pltpu.VMEM((CH, N_NEIGH), jnp.float32),
            pltpu.VMEM((CH,), jnp.float32),
            pltpu.VMEM((CH,), jnp.float32),
        ],
    )
    return run(layer_input, weight, bias)


# speedup vs baseline: 1.8312x; 1.8312x over previous
import functools

import jax
import jax.numpy as jnp
from jax import lax
from jax.experimental import pallas as pl
from jax.experimental.pallas import tpu as pltpu
from jax.experimental.pallas import tpu_sc as plsc

N_NODES = 100000
N_NEIGH = 64
L = 16
NC, NS = 2, 16
NW = NC * NS
CH = 400
NCHUNKS = N_NODES // CH
MAXC = pl.cdiv(NCHUNKS, NW)


def _sc_body(x_hbm, w_hbm, b_hbm, o_hbm, xv, wv, bv, ov):
    wid = lax.axis_index("s") * NC + lax.axis_index("c")
    lane = lax.iota(jnp.int32, L)
    lane64 = lane * N_NEIGH

    def chunk_body(i, carry):
        k = wid + i * NW

        @pl.when(k < NCHUNKS)
        def _():
            base = k * CH
            pltpu.sync_copy(x_hbm.at[pl.ds(base * N_NEIGH, CH * N_NEIGH)], xv)
            pltpu.sync_copy(w_hbm.at[pl.ds(base * N_NEIGH, CH * N_NEIGH)], wv)
            pltpu.sync_copy(b_hbm.at[pl.ds(base, CH)], bv)

            def row_iter(it, c2):
                r0 = it * L
                rows64 = r0 * N_NEIGH + lane64
                accs = [None] * 4
                for j in range(N_NEIGH):
                    colrot = (lane + j) & (N_NEIGH - 1)
                    idx = rows64 + colrot
                    prod = plsc.load_gather(xv, [idx]) * plsc.load_gather(wv, [idx])
                    a = j % 4
                    accs[a] = prod if accs[a] is None else accs[a] + prod
                vec = bv[pl.ds(r0, L)] + ((accs[0] + accs[1]) + (accs[2] + accs[3]))
                ov[pl.ds(r0, L)] = vec
                return c2

            lax.fori_loop(0, CH // L, row_iter, 0)
            pltpu.sync_copy(ov, o_hbm.at[pl.ds(base, CH)])

        return carry

    lax.fori_loop(0, MAXC, chunk_body, 0)


@jax.jit
def kernel(layer_input, weight, bias):
    mesh = plsc.VectorSubcoreMesh(core_axis_name="c", subcore_axis_name="s")
    run = pl.kernel(
        _sc_body,
        out_type=jax.ShapeDtypeStruct((N_NODES,), jnp.float32),
        mesh=mesh,
        compiler_params=pltpu.CompilerParams(
            needs_layout_passes=False,
            use_tc_tiling_on_sc=False,
        ),
        scratch_types=[
            pltpu.VMEM((CH * N_NEIGH,), jnp.float32),
            pltpu.VMEM((CH * N_NEIGH,), jnp.float32),
            pltpu.VMEM((CH,), jnp.float32),
            pltpu.VMEM((CH,), jnp.float32),
        ],
    )
    return run(layer_input.reshape(-1), weight.reshape(-1), bias)


# speedup vs baseline: 2.0570x; 1.1233x over previous
import functools

import jax
import jax.numpy as jnp
from jax import lax
from jax.experimental import pallas as pl
from jax.experimental.pallas import tpu as pltpu
from jax.experimental.pallas import tpu_sc as plsc

N_NODES = 100000
N_NEIGH = 64
L = 16
NC, NS = 2, 16
NW = NC * NS
CH = 400
NCHUNKS = N_NODES // CH
MAXC = pl.cdiv(NCHUNKS, NW)
JB = 16


def _sc_body(x_hbm, w_hbm, b_hbm, o_hbm, xv, wv, bv, ov):
    wid = lax.axis_index("s") * NC + lax.axis_index("c")
    lane = lax.iota(jnp.int32, L)
    lane64 = lane * N_NEIGH

    def chunk_body(i, carry):
        k = wid + i * NW

        @pl.when(k < NCHUNKS)
        def _():
            base = k * CH
            pltpu.sync_copy(x_hbm.at[pl.ds(base * N_NEIGH, CH * N_NEIGH)], xv)
            pltpu.sync_copy(w_hbm.at[pl.ds(base * N_NEIGH, CH * N_NEIGH)], wv)
            pltpu.sync_copy(b_hbm.at[pl.ds(base, CH)], bv)

            def row_iter(it, c2):
                r0 = it * L
                rows64 = r0 * N_NEIGH + lane64

                def col_block(jc, accs):
                    j0 = jc * JB
                    base_col = lane + j0
                    new = list(accs)
                    for j in range(JB):
                        colrot = (base_col + j) & (N_NEIGH - 1)
                        idx = rows64 + colrot
                        prod = plsc.load_gather(xv, [idx]) * plsc.load_gather(
                            wv, [idx]
                        )
                        a = j % 4
                        new[a] = new[a] + prod
                    return tuple(new)

                zeros = jnp.zeros((L,), jnp.float32)
                accs = lax.fori_loop(
                    0, N_NEIGH // JB, col_block, (zeros, zeros, zeros, zeros)
                )
                vec = bv[pl.ds(r0, L)] + ((accs[0] + accs[1]) + (accs[2] + accs[3]))
                ov[pl.ds(r0, L)] = vec
                return c2

            lax.fori_loop(0, CH // L, row_iter, 0)
            pltpu.sync_copy(ov, o_hbm.at[pl.ds(base, CH)])

        return carry

    lax.fori_loop(0, MAXC, chunk_body, 0)


@jax.jit
def kernel(layer_input, weight, bias):
    mesh = plsc.VectorSubcoreMesh(core_axis_name="c", subcore_axis_name="s")
    run = pl.kernel(
        _sc_body,
        out_type=jax.ShapeDtypeStruct((N_NODES,), jnp.float32),
        mesh=mesh,
        compiler_params=pltpu.CompilerParams(
            needs_layout_passes=False,
        ),
        scratch_types=[
            pltpu.VMEM((CH * N_NEIGH,), jnp.float32),
            pltpu.VMEM((CH * N_NEIGH,), jnp.float32),
            pltpu.VMEM((CH,), jnp.float32),
            pltpu.VMEM((CH,), jnp.float32),
        ],
    )
    return run(layer_input.reshape(-1), weight.reshape(-1), bias)


# speedup vs baseline: 2.3977x; 1.1657x over previous
import functools

import jax
import jax.numpy as jnp
from jax import lax
from jax.experimental import pallas as pl
from jax.experimental.pallas import tpu as pltpu
from jax.experimental.pallas import tpu_sc as plsc

N_NODES = 100000
N_NEIGH = 64
L = 16
NC, NS = 2, 16
NW = NC * NS
CH = 400
NCHUNKS = N_NODES // CH
MAXC = pl.cdiv(NCHUNKS, NW)


def _rowsum16(xv, wv, r):
    p0 = xv[r, pl.ds(0, L)] * wv[r, pl.ds(0, L)]
    p1 = xv[r, pl.ds(L, L)] * wv[r, pl.ds(L, L)]
    p2 = xv[r, pl.ds(2 * L, L)] * wv[r, pl.ds(2 * L, L)]
    p3 = xv[r, pl.ds(3 * L, L)] * wv[r, pl.ds(3 * L, L)]
    return (p0 + p1) + (p2 + p3)


def _sc_body(x_hbm, w_hbm, b_hbm, o_hbm, xv, wv, bv, ov):
    wid = lax.axis_index("s") * NC + lax.axis_index("c")
    lane = lax.iota(jnp.int32, L)
    perms = [(lane ^ k)[:, None] for k in (8, 4, 2, 1)]

    def chunk_body(i, carry):
        k = wid + i * NW

        @pl.when(k < NCHUNKS)
        def _():
            base = k * CH
            pltpu.sync_copy(x_hbm.at[pl.ds(base, CH)], xv)
            pltpu.sync_copy(w_hbm.at[pl.ds(base, CH)], wv)
            pltpu.sync_copy(b_hbm.at[pl.ds(base, CH)], bv)

            def row_iter(it, c2):
                r0 = it * L
                vec = bv[pl.ds(r0, L)]
                for u in range(L):
                    s = _rowsum16(xv, wv, r0 + u)
                    for p in perms:
                        s = s + jnp.take_along_axis(s, p[:, 0], axis=0)
                    vec = jnp.where(lane == u, vec + s, vec)
                ov[pl.ds(r0, L)] = vec
                return c2

            lax.fori_loop(0, CH // L, row_iter, 0)
            pltpu.sync_copy(ov, o_hbm.at[pl.ds(base, CH)])

        return carry

    lax.fori_loop(0, MAXC, chunk_body, 0)


@jax.jit
def kernel(layer_input, weight, bias):
    mesh = plsc.VectorSubcoreMesh(core_axis_name="c", subcore_axis_name="s")
    run = pl.kernel(
        _sc_body,
        out_type=jax.ShapeDtypeStruct((N_NODES,), jnp.float32),
        mesh=mesh,
        compiler_params=pltpu.CompilerParams(
            needs_layout_passes=False,
        ),
        scratch_types=[
            pltpu.VMEM((CH, N_NEIGH), jnp.float32),
            pltpu.VMEM((CH, N_NEIGH), jnp.float32),
            pltpu.VMEM((CH,), jnp.float32),
            pltpu.VMEM((CH,), jnp.float32),
        ],
    )
    return run(layer_input, weight, bias)


# speedup vs baseline: 4.2181x; 1.7592x over previous
import functools

import jax
import jax.numpy as jnp
from jax import lax
from jax.experimental import pallas as pl
from jax.experimental.pallas import tpu as pltpu
from jax.experimental.pallas import tpu_sc as plsc

N_NODES = 100000
N_NEIGH = 64
L = 16
NC, NS = 2, 16
NW = NC * NS
CH = 256
NFULL = N_NODES // CH
TAIL = N_NODES - NFULL * CH
MAXC = pl.cdiv(NFULL, NW)
TAIL_WORKER = NW - 2


def _sc_body(x_hbm, w_hbm, b_hbm, xt_hbm, wt_hbm, o_hbm, xv, wv, bv, ov, xvt, wvt):
    wid = lax.axis_index("s") * NC + lax.axis_index("c")

    def compute(xr, wr, nrows):
        def group_iter(it, c2):
            sl = pl.ds(it * L, L)
            accs = [xr[j, sl] * wr[j, sl] for j in range(4)]
            for j in range(4, N_NEIGH):
                accs[j % 4] = accs[j % 4] + xr[j, sl] * wr[j, sl]
            ov[sl] = bv[sl] + ((accs[0] + accs[1]) + (accs[2] + accs[3]))
            return c2

        lax.fori_loop(0, nrows // L, group_iter, 0)

    def chunk_body(i, carry):
        k = wid + i * NW

        @pl.when(k < NFULL)
        def _():
            base = k * CH
            pltpu.sync_copy(x_hbm.at[:, pl.ds(base, CH)], xv)
            pltpu.sync_copy(w_hbm.at[:, pl.ds(base, CH)], wv)
            pltpu.sync_copy(b_hbm.at[pl.ds(base, CH)], bv)
            compute(xv, wv, CH)
            pltpu.sync_copy(ov, o_hbm.at[pl.ds(base, CH)])

        return carry

    lax.fori_loop(0, MAXC, chunk_body, 0)

    @pl.when(wid == TAIL_WORKER)
    def _tail():
        base = NFULL * CH
        pltpu.sync_copy(xt_hbm, xvt)
        pltpu.sync_copy(wt_hbm, wvt)
        pltpu.sync_copy(b_hbm.at[pl.ds(base, TAIL)], bv.at[pl.ds(0, TAIL)])
        compute(xvt, wvt, TAIL)
        pltpu.sync_copy(ov.at[pl.ds(0, TAIL)], o_hbm.at[pl.ds(base, TAIL)])


@jax.jit
def kernel(layer_input, weight, bias):
    mesh = plsc.VectorSubcoreMesh(core_axis_name="c", subcore_axis_name="s")
    run = pl.kernel(
        _sc_body,
        out_type=jax.ShapeDtypeStruct((N_NODES,), jnp.float32),
        mesh=mesh,
        compiler_params=pltpu.CompilerParams(
            needs_layout_passes=False,
        ),
        scratch_types=[
            pltpu.VMEM((N_NEIGH, CH), jnp.float32),
            pltpu.VMEM((N_NEIGH, CH), jnp.float32),
            pltpu.VMEM((CH,), jnp.float32),
            pltpu.VMEM((CH,), jnp.float32),
            pltpu.VMEM((N_NEIGH, TAIL), jnp.float32),
            pltpu.VMEM((N_NEIGH, TAIL), jnp.float32),
        ],
    )
    xtail = lax.slice(layer_input, (NFULL * CH, 0), (N_NODES, N_NEIGH)).T
    wtail = lax.slice(weight, (NFULL * CH, 0), (N_NODES, N_NEIGH)).T
    return run(layer_input.T, weight.T, bias, xtail, wtail)


# speedup vs baseline: 7.0806x; 1.6786x over previous
import functools

import jax
import jax.numpy as jnp
from jax import lax
from jax.experimental import pallas as pl
from jax.experimental.pallas import tpu as pltpu
from jax.experimental.pallas import tpu_sc as plsc

N_NODES = 100000
N_NEIGH = 64
L = 16
NC, NS = 2, 16
NW = NC * NS
CH = 256
NFULL = N_NODES // CH
TAIL = N_NODES - NFULL * CH
MAXC = pl.cdiv(NFULL, NW)
NPAIR = (MAXC + 1) // 2
TAIL_WORKER = NW - 2


def _sc_body(
    x_hbm, w_hbm, b_hbm, xt_hbm, wt_hbm, o_hbm,
    xv0, wv0, bv0, ov0, xv1, wv1, bv1, ov1, xvt, wvt, bvt, ovt,
    sin0, sin1, sout0, sout1, stail,
):
    wid = lax.axis_index("s") * NC + lax.axis_index("c")
    xvs, wvs, bvs, ovs = (xv0, xv1), (wv0, wv1), (bv0, bv1), (ov0, ov1)
    sins, souts = (sin0, sin1), (sout0, sout1)

    def start_in(k, b):
        base = k * CH
        pltpu.async_copy(x_hbm.at[:, pl.ds(base, CH)], xvs[b], sins[b])
        pltpu.async_copy(w_hbm.at[:, pl.ds(base, CH)], wvs[b], sins[b])
        pltpu.async_copy(b_hbm.at[pl.ds(base, CH)], bvs[b], sins[b])

    def wait_in(k, b):
        base = k * CH
        pltpu.make_async_copy(x_hbm.at[:, pl.ds(base, CH)], xvs[b], sins[b]).wait()
        pltpu.make_async_copy(w_hbm.at[:, pl.ds(base, CH)], wvs[b], sins[b]).wait()
        pltpu.make_async_copy(b_hbm.at[pl.ds(base, CH)], bvs[b], sins[b]).wait()

    def start_out(k, b):
        pltpu.async_copy(ovs[b], o_hbm.at[pl.ds(k * CH, CH)], souts[b])

    def wait_out(k, b):
        pltpu.make_async_copy(ovs[b], o_hbm.at[pl.ds(k * CH, CH)], souts[b]).wait()

    def compute(xr, wr, br, orr, nrows):
        def group_iter(it, c2):
            sl = pl.ds(it * L, L)
            accs = [xr[j, sl] * wr[j, sl] for j in range(4)]
            for j in range(4, N_NEIGH):
                accs[j % 4] = accs[j % 4] + xr[j, sl] * wr[j, sl]
            orr[sl] = br[sl] + ((accs[0] + accs[1]) + (accs[2] + accs[3]))
            return c2

        lax.fori_loop(0, nrows // L, group_iter, 0)

    @pl.when(wid == TAIL_WORKER)
    def _tail_start():
        pltpu.async_copy(xt_hbm, xvt, stail)
        pltpu.async_copy(wt_hbm, wvt, stail)
        pltpu.async_copy(b_hbm.at[pl.ds(NFULL * CH, TAIL)], bvt, stail)

    start_in(wid, 0)

    def pair_body(p, carry):
        i0 = 2 * p
        i1 = i0 + 1
        k0 = wid + i0 * NW
        k1 = wid + i1 * NW

        @pl.when(k1 < NFULL)
        def _():
            start_in(k1, 1)

        @pl.when(k0 < NFULL)
        def _():
            wait_in(k0, 0)

            @pl.when(i0 >= 2)
            def _():
                wait_out(k0 - 2 * NW, 0)

            compute(xvs[0], wvs[0], bvs[0], ovs[0], CH)
            start_out(k0, 0)

        @pl.when(k1 + NW < NFULL)
        def _():
            start_in(k1 + NW, 0)

        @pl.when(k1 < NFULL)
        def _():
            wait_in(k1, 1)

            @pl.when(i1 >= 3)
            def _():
                wait_out(k1 - 2 * NW, 1)

            compute(xvs[1], wvs[1], bvs[1], ovs[1], CH)
            start_out(k1, 1)

        return carry

    lax.fori_loop(0, NPAIR, pair_body, 0)

    nch = (NFULL - wid + NW - 1) // NW
    ilast = nch - 1
    klast = wid + ilast * NW

    @pl.when(ilast % 2 == 0)
    def _():
        wait_out(klast, 0)
        wait_out(klast - NW, 1)

    @pl.when(ilast % 2 == 1)
    def _():
        wait_out(klast, 1)
        wait_out(klast - NW, 0)

    @pl.when(wid == TAIL_WORKER)
    def _tail():
        base = NFULL * CH
        pltpu.make_async_copy(xt_hbm, xvt, stail).wait()
        pltpu.make_async_copy(wt_hbm, wvt, stail).wait()
        pltpu.make_async_copy(
            b_hbm.at[pl.ds(base, TAIL)], bvt, stail
        ).wait()
        compute(xvt, wvt, bvt, ovt, TAIL)
        pltpu.sync_copy(ovt, o_hbm.at[pl.ds(base, TAIL)])


@jax.jit
def kernel(layer_input, weight, bias):
    mesh = plsc.VectorSubcoreMesh(core_axis_name="c", subcore_axis_name="s")
    run = pl.kernel(
        _sc_body,
        out_type=jax.ShapeDtypeStruct((N_NODES,), jnp.float32),
        mesh=mesh,
        compiler_params=pltpu.CompilerParams(
            needs_layout_passes=False,
        ),
        scratch_types=[
            pltpu.VMEM((N_NEIGH, CH), jnp.float32),
            pltpu.VMEM((N_NEIGH, CH), jnp.float32),
            pltpu.VMEM((CH,), jnp.float32),
            pltpu.VMEM((CH,), jnp.float32),
            pltpu.VMEM((N_NEIGH, CH), jnp.float32),
            pltpu.VMEM((N_NEIGH, CH), jnp.float32),
            pltpu.VMEM((CH,), jnp.float32),
            pltpu.VMEM((CH,), jnp.float32),
            pltpu.VMEM((N_NEIGH, TAIL), jnp.float32),
            pltpu.VMEM((N_NEIGH, TAIL), jnp.float32),
            pltpu.VMEM((TAIL,), jnp.float32),
            pltpu.VMEM((TAIL,), jnp.float32),
            pltpu.SemaphoreType.DMA,
            pltpu.SemaphoreType.DMA,
            pltpu.SemaphoreType.DMA,
            pltpu.SemaphoreType.DMA,
            pltpu.SemaphoreType.DMA,
        ],
    )
    xtail = lax.slice(layer_input, (NFULL * CH, 0), (N_NODES, N_NEIGH)).T
    wtail = lax.slice(weight, (NFULL * CH, 0), (N_NODES, N_NEIGH)).T
    return run(layer_input.T, weight.T, bias, xtail, wtail)


# speedup vs baseline: 15.2187x; 2.1494x over previous
import jax
import jax.numpy as jnp
from jax.experimental import pallas as pl

N_NODES = 100000
N_NEIGH = 64
BT = 4096


def _body(x_ref, w_ref, b_ref, o_ref):
    o_ref[...] = b_ref[...] + jnp.sum(x_ref[...] * w_ref[...], axis=0)


@jax.jit
def kernel(layer_input, weight, bias):
    xT = layer_input.T
    wT = weight.T
    grid = (pl.cdiv(N_NODES, BT),)
    return pl.pallas_call(
        _body,
        grid=grid,
        in_specs=[
            pl.BlockSpec((N_NEIGH, BT), lambda i: (0, i)),
            pl.BlockSpec((N_NEIGH, BT), lambda i: (0, i)),
            pl.BlockSpec((BT,), lambda i: (i,)),
        ],
        out_specs=pl.BlockSpec((BT,), lambda i: (i,)),
        out_shape=jax.ShapeDtypeStruct((N_NODES,), jnp.float32),
    )(xT, wT, bias)
